# baseline (speedup 1.0000x reference)
import functools

import jax
import jax.numpy as jnp
from jax import lax
from jax.experimental import pallas as pl
from jax.experimental.pallas import tpu as pltpu
from jax.experimental.pallas import tpu_sc as plsc

N = 10000
E = 320000
R = 8
F_IN = 128
H1 = 64
H2 = 32

NC = 2
NS = 16
NW = NC * NS

NPAD = 10112
ROWS_PER_TILE = 632
CNT = N * R
CNT_PER_TILE = CNT // NS

E_CNT_TILE = E // NS
E_OUT_TILE = E // NW
CH = 80


def _sc_mesh():
    return plsc.VectorSubcoreMesh(core_axis_name="c", subcore_axis_name="s",
                                  num_cores=NC, num_subcores=NS)


def _norm_body(src_hbm, dst_hbm, et_hbm,
               g_hbm, norm_hbm,
               cnt_sh, ccopy, dbuf, ebuf, sbuf, pairbuf, gbuf, nbuf, ones, sem):
    c = lax.axis_index("c")
    s = lax.axis_index("s")

    @plsc.parallel_loop(0, (CNT_PER_TILE + 15) // 16, 1, unroll=8)
    def _(i):
        ccopy[pl.ds(16 * i, 16)] = jnp.zeros((16,), jnp.float32)
    pltpu.sync_copy(ccopy.at[pl.ds(0, CNT_PER_TILE)],
                    cnt_sh.at[pl.ds(s * CNT_PER_TILE, CNT_PER_TILE)])
    for j in range(CH // 16):
        ones[pl.ds(16 * j, 16)] = jnp.ones((16,), jnp.float32)
    plsc.subcore_barrier()

    OC = 4000
    NIN = OC // CH

    def count_outer(o, _):
        base = s * E_CNT_TILE + o * OC
        pltpu.sync_copy(dst_hbm.at[pl.ds(base, OC)], dbuf)
        pltpu.sync_copy(et_hbm.at[pl.ds(base, OC)], ebuf)

        @plsc.parallel_loop(0, NIN, 1, unroll=4)
        def _(i):
            for j in range(CH // 16):
                dv = dbuf[pl.ds(i * CH + 16 * j, 16)]
                ev = ebuf[pl.ds(i * CH + 16 * j, 16)]
                pairbuf[i, pl.ds(16 * j, 16)] = dv * R + ev
        descs = [pltpu.async_copy(ones, cnt_sh.at[pairbuf.at[i]], sem,
                                  add=True) for i in range(NIN)]
        for d in descs:
            d.wait()
        return 0
    lax.fori_loop(0, E_CNT_TILE // OC, count_outer, 0)
    plsc.subcore_barrier()

    pltpu.sync_copy(cnt_sh.at[pl.ds(s * CNT_PER_TILE, CNT_PER_TILE)],
                    ccopy.at[pl.ds(0, CNT_PER_TILE)])

    @plsc.parallel_loop(0, (CNT_PER_TILE + 15) // 16, 1, unroll=8)
    def _(i):
        ccopy[pl.ds(16 * i, 16)] = 1.0 / ccopy[pl.ds(16 * i, 16)]
    pltpu.sync_copy(ccopy.at[pl.ds(0, CNT_PER_TILE)],
                    cnt_sh.at[pl.ds(s * CNT_PER_TILE, CNT_PER_TILE)])
    plsc.subcore_barrier()

    pltpu.sync_copy(cnt_sh, ccopy)

    OC2 = 2000
    wid = s * NC + c

    def norm_outer(o, _):
        base = wid * E_OUT_TILE + o * OC2
        pltpu.sync_copy(src_hbm.at[pl.ds(base, OC2)], sbuf)
        pltpu.sync_copy(dst_hbm.at[pl.ds(base, OC2)], dbuf.at[pl.ds(0, OC2)])
        pltpu.sync_copy(et_hbm.at[pl.ds(base, OC2)], ebuf.at[pl.ds(0, OC2)])

        @plsc.parallel_loop(0, OC2 // 16, 1, unroll=4)
        def _(i):
            sv = sbuf[pl.ds(16 * i, 16)]
            dv = dbuf[pl.ds(16 * i, 16)]
            ev = ebuf[pl.ds(16 * i, 16)]
            gbuf[pl.ds(16 * i, 16)] = sv * (R + 1) + ev
            nbuf[pl.ds(16 * i, 16)] = plsc.load_gather(ccopy, [dv * R + ev])
        pltpu.sync_copy(gbuf, g_hbm.at[pl.ds(base, OC2)])
        pltpu.sync_copy(nbuf, norm_hbm.at[pl.ds(base, OC2)])
        return 0
    lax.fori_loop(0, E_OUT_TILE // OC2, norm_outer, 0)


def _sc_norm(src, dst, et):
    kern = pl.kernel(
        _norm_body,
        out_type=(jax.ShapeDtypeStruct((E,), jnp.int32),
                  jax.ShapeDtypeStruct((E,), jnp.float32)),
        mesh=_sc_mesh(),
        scratch_types=[
            pltpu.VMEM_SHARED((CNT,), jnp.float32),
            pltpu.VMEM((CNT,), jnp.float32),
            pltpu.VMEM((4000,), jnp.int32),
            pltpu.VMEM((4000,), jnp.int32),
            pltpu.VMEM((2000,), jnp.int32),
            pltpu.VMEM((50, CH), jnp.int32),
            pltpu.VMEM((2000,), jnp.int32),
            pltpu.VMEM((2000,), jnp.float32),
            pltpu.VMEM((CH,), jnp.float32),
            pltpu.SemaphoreType.DMA,
        ],
        compiler_params=pltpu.CompilerParams(needs_layout_passes=False, use_tc_tiling_on_sc=False),
    )
    return kern(src, dst, et)


def _agg_body(h, table_hbm, g_hbm, dst_hbm, norm_hbm, out_hbm,
              agg_sh, gflat, dstflat, dstbuf, nbuf, rows0, rows1, rows2, zbuf,
              gsem0, gsem1, gsem2, ssem0, ssem1, ssem2):
    c = lax.axis_index("c")
    s = lax.axis_index("s")
    wid = s * NC + c
    rows = (rows0, rows1, rows2)
    gsems = (gsem0, gsem1, gsem2)
    ssems = (ssem0, ssem1, ssem2)
    NBUF = 3

    @plsc.parallel_loop(0, ROWS_PER_TILE, 1, unroll=8)
    def _(i):
        for j in range(h // 16):
            zbuf[i, pl.ds(16 * j, 16)] = jnp.zeros((16,), jnp.float32)
    pltpu.sync_copy(zbuf, agg_sh.at[pl.ds(s * ROWS_PER_TILE, ROWS_PER_TILE)])
    plsc.subcore_barrier()

    OC = 2000
    NIN = OC // CH

    def outer(o, _):
        base = wid * E_OUT_TILE + o * OC
        pltpu.sync_copy(g_hbm.at[pl.ds(base, OC)], gflat)
        pltpu.sync_copy(dst_hbm.at[pl.ds(base, OC)], dstflat)
        pltpu.sync_copy(norm_hbm.at[pl.ds(base, OC)], nbuf)

        @plsc.parallel_loop(0, NIN, 1, unroll=4)
        def _(i):
            for j in range(CH // 16):
                dstbuf[i, pl.ds(16 * j, 16)] = dstflat[pl.ds(i * CH + 16 * j, 16)]

        def fire_gather(i):
            return pltpu.async_copy(
                table_hbm.at[gflat.at[pl.ds(i * CH, CH)]], rows[i % NBUF],
                gsems[i % NBUF])

        gd = [None] * NIN
        sd = [None] * NIN
        gd[0] = fire_gather(0)
        for i in range(NIN):
            b = i % NBUF
            if i + 1 < NIN:
                if i + 1 >= NBUF:
                    sd[i + 1 - NBUF].wait()
                gd[i + 1] = fire_gather(i + 1)
            gd[i].wait()

            @plsc.parallel_loop(0, CH, 1, unroll=16)
            def _(k, _i=i, _b=b):
                nv = plsc.load_gather(
                    nbuf, [jnp.full((16,), _i * CH, jnp.int32) + k])
                for j in range(h // 16):
                    rows[_b][k, pl.ds(16 * j, 16)] = (
                        rows[_b][k, pl.ds(16 * j, 16)] * nv)
            sd[i] = pltpu.async_copy(rows[b], agg_sh.at[dstbuf.at[i]],
                                     ssems[b], add=True)
        for i in range(NIN - NBUF, NIN):
            sd[i].wait()
        return 0
    lax.fori_loop(0, E_OUT_TILE // OC, outer, 0)
    plsc.subcore_barrier()

    r0 = s * ROWS_PER_TILE
    LAST = N - (NS - 1) * ROWS_PER_TILE

    @pl.when(s < NS - 1)
    def _():
        pltpu.sync_copy(agg_sh.at[pl.ds(r0, ROWS_PER_TILE)], zbuf)
        pltpu.sync_copy(zbuf, out_hbm.at[c].at[pl.ds(r0, ROWS_PER_TILE)])

    @pl.when(s == NS - 1)
    def _():
        pltpu.sync_copy(agg_sh.at[pl.ds(r0, LAST)], zbuf.at[pl.ds(0, LAST)])
        pltpu.sync_copy(zbuf.at[pl.ds(0, LAST)],
                        out_hbm.at[c].at[pl.ds(r0, LAST)])


def _sc_agg(h, table, g, dst, norm):
    kern = pl.kernel(
        functools.partial(_agg_body, h),
        out_type=jax.ShapeDtypeStruct((NC, N, h), jnp.float32),
        mesh=_sc_mesh(),
        scratch_types=[
            pltpu.VMEM_SHARED((NPAD, h), jnp.float32),
            pltpu.VMEM((2000,), jnp.int32),
            pltpu.VMEM((2000,), jnp.int32),
            pltpu.VMEM((25, CH), jnp.int32),
            pltpu.VMEM((2000,), jnp.float32),
            pltpu.VMEM((CH, h), jnp.float32),
            pltpu.VMEM((CH, h), jnp.float32),
            pltpu.VMEM((CH, h), jnp.float32),
            pltpu.VMEM((ROWS_PER_TILE, h), jnp.float32),
            pltpu.SemaphoreType.DMA,
            pltpu.SemaphoreType.DMA,
            pltpu.SemaphoreType.DMA,
            pltpu.SemaphoreType.DMA,
            pltpu.SemaphoreType.DMA,
            pltpu.SemaphoreType.DMA,
        ],
        compiler_params=pltpu.CompilerParams(needs_layout_passes=False, use_tc_tiling_on_sc=False),
    )
    return kern(table, g, dst, norm)


BN = 2000


def _mm_body(x_ref, w_ref, b_ref, o_ref):
    o_ref[...] = jnp.dot(x_ref[...], w_ref[...],
                         preferred_element_type=jnp.float32) + b_ref[...]


def _tc_mm(x, w, brow):
    k = x.shape[1]
    m = w.shape[1]
    return pl.pallas_call(
        _mm_body,
        grid=(N // BN,),
        in_specs=[pl.BlockSpec((BN, k), lambda i: (i, 0)),
                  pl.BlockSpec((k, m), lambda i: (0, 0)),
                  pl.BlockSpec((1, m), lambda i: (0, 0))],
        out_specs=pl.BlockSpec((BN, m), lambda i: (i, 0)),
        out_shape=jax.ShapeDtypeStruct((N, m), jnp.float32),
    )(x, w, brow)


def _mid_body(agg_ref, xroot_ref, w_ref, b_ref, h1_ref, o_ref):
    h1 = jnp.maximum(agg_ref[0] + agg_ref[1] + xroot_ref[...], 0.0)
    h1_ref[...] = h1
    o_ref[...] = jnp.dot(h1, w_ref[...],
                         preferred_element_type=jnp.float32) + b_ref[...]


def _tc_mid(aggpair, projall, w, brow):
    k = w.shape[0]
    m = w.shape[1]
    return pl.pallas_call(
        _mid_body,
        grid=(N // BN,),
        in_specs=[pl.BlockSpec((NC, BN, k), lambda i: (0, i, 0)),
                  pl.BlockSpec((BN, k), lambda i: (i, 0)),
                  pl.BlockSpec((k, m), lambda i: (0, 0)),
                  pl.BlockSpec((1, m), lambda i: (0, 0))],
        out_specs=[pl.BlockSpec((BN, k), lambda i: (i, 0)),
                   pl.BlockSpec((BN, m), lambda i: (i, 0))],
        out_shape=[jax.ShapeDtypeStruct((N, k), jnp.float32),
                   jax.ShapeDtypeStruct((N, m), jnp.float32)],
    )(aggpair, projall, w, brow)


def _head_body(h1_ref, agg_ref, hroot_ref, fc1_wT_ref, fc1_b_ref,
               w_ihT_ref, bsum_ref, fc2_wT_ref, fc2_b_ref, o_ref,
               acc1, acc2):
    i = pl.program_id(0)

    @pl.when(i == 0)
    def _():
        acc1[...] = jnp.zeros_like(acc1)
        acc2[...] = jnp.zeros_like(acc2)

    h2 = jnp.maximum(agg_ref[0] + agg_ref[1] + hroot_ref[...], 0.0)
    acc1[...] += jnp.sum(h1_ref[...], axis=0, keepdims=True)
    acc2[...] += jnp.sum(h2, axis=0, keepdims=True)

    @pl.when(i == N // BN - 1)
    def _():
        pooled = jnp.concatenate([acc1[...], acc2[...]], axis=1) / float(N)
        f1 = jnp.maximum(
            jnp.dot(pooled, fc1_wT_ref[...],
                    preferred_element_type=jnp.float32) + fc1_b_ref[...], 0.0)
        gates = jnp.dot(f1, w_ihT_ref[...],
                        preferred_element_type=jnp.float32) + bsum_ref[...]
        hd = 20
        i_g = jax.nn.sigmoid(gates[:, :hd])
        g_g = jnp.tanh(gates[:, 2 * hd:3 * hd])
        o_g = jax.nn.sigmoid(gates[:, 3 * hd:4 * hd])
        hvec = o_g * jnp.tanh(i_g * g_g)
        logits = jnp.dot(hvec, fc2_wT_ref[...],
                         preferred_element_type=jnp.float32) + fc2_b_ref[...]
        m = jnp.max(logits, axis=1, keepdims=True)
        lse = m + jnp.log(jnp.sum(jnp.exp(logits - m), axis=1, keepdims=True))
        o_ref[...] = logits - lse


def _tc_head(h1, aggpair, hroot, fc1_wT, fc1_b, w_ihT, bsum, fc2_wT, fc2_b):
    return pl.pallas_call(
        _head_body,
        grid=(N // BN,),
        in_specs=[pl.BlockSpec((BN, H1), lambda i: (i, 0)),
                  pl.BlockSpec((NC, BN, H2), lambda i: (0, i, 0)),
                  pl.BlockSpec((BN, H2), lambda i: (i, 0)),
                  pl.BlockSpec(fc1_wT.shape, lambda i: (0, 0)),
                  pl.BlockSpec(fc1_b.shape, lambda i: (0, 0)),
                  pl.BlockSpec(w_ihT.shape, lambda i: (0, 0)),
                  pl.BlockSpec(bsum.shape, lambda i: (0, 0)),
                  pl.BlockSpec(fc2_wT.shape, lambda i: (0, 0)),
                  pl.BlockSpec(fc2_b.shape, lambda i: (0, 0))],
        out_specs=pl.BlockSpec((1, 10), lambda i: (0, 0)),
        out_shape=jax.ShapeDtypeStruct((1, 10), jnp.float32),
        scratch_shapes=[pltpu.VMEM((1, H1), jnp.float32),
                        pltpu.VMEM((1, H2), jnp.float32)],
    )(h1, aggpair, hroot, fc1_wT, fc1_b, w_ihT, bsum, fc2_wT, fc2_b)


def kernel(x, edge_index, edge_attr, batch, w1, root1, b1, w2, root2, b2,
           fc1_w, fc1_b, w_ih, w_hh, b_ih, b_hh, fc2_w, fc2_b):
    src = edge_index[0]
    dst = edge_index[1]
    et = edge_attr

    wcat1 = jnp.concatenate(
        [jnp.transpose(w1, (1, 0, 2)).reshape(F_IN, R * H1), root1], axis=1)
    brow1 = jnp.concatenate([jnp.zeros((R * H1,), jnp.float32), b1])[None, :]
    wcat2 = jnp.concatenate(
        [jnp.transpose(w2, (1, 0, 2)).reshape(H1, R * H2), root2], axis=1)
    brow2 = jnp.concatenate([jnp.zeros((R * H2,), jnp.float32), b2])[None, :]

    g, norm = _sc_norm(src, dst, et)

    projall1 = _tc_mm(x, wcat1, brow1)
    table1 = projall1.reshape(N * (R + 1), H1)
    agg1 = _sc_agg(H1, table1, g, dst, norm)

    h1, projall2 = _tc_mid(agg1, projall1[:, R * H1:], wcat2, brow2)
    table2 = projall2.reshape(N * (R + 1), H2)
    agg2 = _sc_agg(H2, table2, g, dst, norm)

    out = _tc_head(h1, agg2, projall2[:, R * H2:],
                   fc1_w.T, fc1_b[None, :], w_ih.T,
                   (b_ih + b_hh)[None, :], fc2_w.T, fc2_b[None, :])
    return out.reshape((10,))


# speedup vs baseline: 1.0879x; 1.0879x over previous
import functools

import jax
import jax.numpy as jnp
from jax import lax
from jax.experimental import pallas as pl
from jax.experimental.pallas import tpu as pltpu
from jax.experimental.pallas import tpu_sc as plsc

N = 10000
E = 320000
R = 8
F_IN = 128
H1 = 64
H2 = 32

NC = 2
NS = 16
NW = NC * NS

NPAD = 10112
ROWS_PER_TILE = 632
CNT = N * R
CNT_PER_TILE = CNT // NS

E_CNT_TILE = E // NS
E_OUT_TILE = E // NW
CH = 80


def _sc_mesh():
    return plsc.VectorSubcoreMesh(core_axis_name="c", subcore_axis_name="s",
                                  num_cores=NC, num_subcores=NS)


def _norm_body(src_hbm, dst_hbm, et_hbm,
               g_hbm, norm_hbm,
               cnt_sh, ccopy, dbuf, ebuf, sbuf, pairbuf, gbuf, nbuf, ones, sem):
    c = lax.axis_index("c")
    s = lax.axis_index("s")

    @plsc.parallel_loop(0, (CNT_PER_TILE + 15) // 16, 1, unroll=8)
    def _(i):
        ccopy[pl.ds(16 * i, 16)] = jnp.zeros((16,), jnp.float32)
    pltpu.sync_copy(ccopy.at[pl.ds(0, CNT_PER_TILE)],
                    cnt_sh.at[pl.ds(s * CNT_PER_TILE, CNT_PER_TILE)])
    for j in range(CH // 16):
        ones[pl.ds(16 * j, 16)] = jnp.ones((16,), jnp.float32)
    plsc.subcore_barrier()

    OC = 4000
    NIN = OC // CH

    def count_outer(o, _):
        base = s * E_CNT_TILE + o * OC
        pltpu.sync_copy(dst_hbm.at[pl.ds(base, OC)], dbuf)
        pltpu.sync_copy(et_hbm.at[pl.ds(base, OC)], ebuf)

        @plsc.parallel_loop(0, NIN, 1, unroll=4)
        def _(i):
            for j in range(CH // 16):
                dv = dbuf[pl.ds(i * CH + 16 * j, 16)]
                ev = ebuf[pl.ds(i * CH + 16 * j, 16)]
                pairbuf[i, pl.ds(16 * j, 16)] = dv * R + ev
        descs = [pltpu.async_copy(ones, cnt_sh.at[pairbuf.at[i]], sem,
                                  add=True) for i in range(NIN)]
        for d in descs:
            d.wait()
        return 0
    lax.fori_loop(0, E_CNT_TILE // OC, count_outer, 0)
    plsc.subcore_barrier()

    pltpu.sync_copy(cnt_sh.at[pl.ds(s * CNT_PER_TILE, CNT_PER_TILE)],
                    ccopy.at[pl.ds(0, CNT_PER_TILE)])

    @plsc.parallel_loop(0, (CNT_PER_TILE + 15) // 16, 1, unroll=8)
    def _(i):
        ccopy[pl.ds(16 * i, 16)] = 1.0 / ccopy[pl.ds(16 * i, 16)]
    pltpu.sync_copy(ccopy.at[pl.ds(0, CNT_PER_TILE)],
                    cnt_sh.at[pl.ds(s * CNT_PER_TILE, CNT_PER_TILE)])
    plsc.subcore_barrier()

    pltpu.sync_copy(cnt_sh, ccopy)

    OC2 = 2000
    wid = s * NC + c

    def norm_outer(o, _):
        base = wid * E_OUT_TILE + o * OC2
        pltpu.sync_copy(src_hbm.at[pl.ds(base, OC2)], sbuf)
        pltpu.sync_copy(dst_hbm.at[pl.ds(base, OC2)], dbuf.at[pl.ds(0, OC2)])
        pltpu.sync_copy(et_hbm.at[pl.ds(base, OC2)], ebuf.at[pl.ds(0, OC2)])

        @plsc.parallel_loop(0, OC2 // 16, 1, unroll=4)
        def _(i):
            sv = sbuf[pl.ds(16 * i, 16)]
            dv = dbuf[pl.ds(16 * i, 16)]
            ev = ebuf[pl.ds(16 * i, 16)]
            gbuf[pl.ds(16 * i, 16)] = sv * (R + 1) + ev
            nbuf[pl.ds(16 * i, 16)] = plsc.load_gather(ccopy, [dv * R + ev])
        pltpu.sync_copy(gbuf, g_hbm.at[pl.ds(base, OC2)])
        pltpu.sync_copy(nbuf, norm_hbm.at[pl.ds(base, OC2)])
        return 0
    lax.fori_loop(0, E_OUT_TILE // OC2, norm_outer, 0)


def _sc_norm(src, dst, et):
    kern = pl.kernel(
        _norm_body,
        out_type=(jax.ShapeDtypeStruct((E,), jnp.int32),
                  jax.ShapeDtypeStruct((E,), jnp.float32)),
        mesh=_sc_mesh(),
        scratch_types=[
            pltpu.VMEM_SHARED((CNT,), jnp.float32),
            pltpu.VMEM((CNT,), jnp.float32),
            pltpu.VMEM((4000,), jnp.int32),
            pltpu.VMEM((4000,), jnp.int32),
            pltpu.VMEM((2000,), jnp.int32),
            pltpu.VMEM((50, CH), jnp.int32),
            pltpu.VMEM((2000,), jnp.int32),
            pltpu.VMEM((2000,), jnp.float32),
            pltpu.VMEM((CH,), jnp.float32),
            pltpu.SemaphoreType.DMA,
        ],
        compiler_params=pltpu.CompilerParams(needs_layout_passes=False, use_tc_tiling_on_sc=False),
    )
    return kern(src, dst, et)


def _agg_body(h, table_hbm, g_hbm, dst_hbm, norm_hbm, out_hbm,
              agg_sh, gflat, dstflat, dstbuf, nbuf, rows0, rows1, rows2,
              rows3, zbuf,
              gsem0, gsem1, gsem2, gsem3, ssem0, ssem1, ssem2, ssem3):
    c = lax.axis_index("c")
    s = lax.axis_index("s")
    wid = s * NC + c
    rows = (rows0, rows1, rows2, rows3)
    gsems = (gsem0, gsem1, gsem2, gsem3)
    ssems = (ssem0, ssem1, ssem2, ssem3)
    NBUF = 4

    @plsc.parallel_loop(0, ROWS_PER_TILE, 1, unroll=8)
    def _(i):
        for j in range(h // 16):
            zbuf[i, pl.ds(16 * j, 16)] = jnp.zeros((16,), jnp.float32)
    pltpu.sync_copy(zbuf, agg_sh.at[pl.ds(s * ROWS_PER_TILE, ROWS_PER_TILE)])
    plsc.subcore_barrier()

    OC = 2000
    NIN = OC // CH

    def outer(o, _):
        base = wid * E_OUT_TILE + o * OC
        pltpu.sync_copy(g_hbm.at[pl.ds(base, OC)], gflat)
        pltpu.sync_copy(dst_hbm.at[pl.ds(base, OC)], dstflat)
        pltpu.sync_copy(norm_hbm.at[pl.ds(base, OC)], nbuf)

        @plsc.parallel_loop(0, NIN, 1, unroll=4)
        def _(i):
            for j in range(CH // 16):
                dstbuf[i, pl.ds(16 * j, 16)] = dstflat[pl.ds(i * CH + 16 * j, 16)]

        def fire_gather(i):
            return pltpu.async_copy(
                table_hbm.at[gflat.at[pl.ds(i * CH, CH)]], rows[i % NBUF],
                gsems[i % NBUF])

        gd = [None] * NIN
        sd = [None] * NIN
        gd[0] = fire_gather(0)
        for i in range(NIN):
            b = i % NBUF
            if i + 1 < NIN:
                if i + 1 >= NBUF:
                    sd[i + 1 - NBUF].wait()
                gd[i + 1] = fire_gather(i + 1)
            gd[i].wait()

            @plsc.parallel_loop(0, CH, 1, unroll=8)
            def _(k, _i=i, _b=b):
                nv = plsc.load_gather(
                    nbuf, [jnp.full((16,), _i * CH, jnp.int32) + k])
                for j in range(h // 16):
                    rows[_b][k, pl.ds(16 * j, 16)] = (
                        rows[_b][k, pl.ds(16 * j, 16)] * nv)
            sd[i] = pltpu.async_copy(rows[b], agg_sh.at[dstbuf.at[i]],
                                     ssems[b], add=True)
        for i in range(NIN - NBUF, NIN):
            sd[i].wait()
        return 0
    lax.fori_loop(0, E_OUT_TILE // OC, outer, 0)
    plsc.subcore_barrier()

    r0 = s * ROWS_PER_TILE
    LAST = N - (NS - 1) * ROWS_PER_TILE

    @pl.when(s < NS - 1)
    def _():
        pltpu.sync_copy(agg_sh.at[pl.ds(r0, ROWS_PER_TILE)], zbuf)
        pltpu.sync_copy(zbuf, out_hbm.at[c].at[pl.ds(r0, ROWS_PER_TILE)])

    @pl.when(s == NS - 1)
    def _():
        pltpu.sync_copy(agg_sh.at[pl.ds(r0, LAST)], zbuf.at[pl.ds(0, LAST)])
        pltpu.sync_copy(zbuf.at[pl.ds(0, LAST)],
                        out_hbm.at[c].at[pl.ds(r0, LAST)])


def _sc_agg(h, table, g, dst, norm):
    kern = pl.kernel(
        functools.partial(_agg_body, h),
        out_type=jax.ShapeDtypeStruct((NC, N, h), jnp.float32),
        mesh=_sc_mesh(),
        scratch_types=[
            pltpu.VMEM_SHARED((NPAD, h), jnp.float32),
            pltpu.VMEM((2000,), jnp.int32),
            pltpu.VMEM((2000,), jnp.int32),
            pltpu.VMEM((25, CH), jnp.int32),
            pltpu.VMEM((2000,), jnp.float32),
            pltpu.VMEM((CH, h), jnp.float32),
            pltpu.VMEM((CH, h), jnp.float32),
            pltpu.VMEM((CH, h), jnp.float32),
            pltpu.VMEM((CH, h), jnp.float32),
            pltpu.VMEM((ROWS_PER_TILE, h), jnp.float32),
            pltpu.SemaphoreType.DMA,
            pltpu.SemaphoreType.DMA,
            pltpu.SemaphoreType.DMA,
            pltpu.SemaphoreType.DMA,
            pltpu.SemaphoreType.DMA,
            pltpu.SemaphoreType.DMA,
            pltpu.SemaphoreType.DMA,
            pltpu.SemaphoreType.DMA,
        ],
        compiler_params=pltpu.CompilerParams(needs_layout_passes=False, use_tc_tiling_on_sc=False),
    )
    return kern(table, g, dst, norm)


BN = 2000


def _mm_body(x_ref, w_ref, b_ref, o_ref):
    o_ref[...] = jnp.dot(x_ref[...], w_ref[...],
                         preferred_element_type=jnp.float32) + b_ref[...]


def _tc_mm(x, w, brow):
    k = x.shape[1]
    m = w.shape[1]
    return pl.pallas_call(
        _mm_body,
        grid=(N // BN,),
        in_specs=[pl.BlockSpec((BN, k), lambda i: (i, 0)),
                  pl.BlockSpec((k, m), lambda i: (0, 0)),
                  pl.BlockSpec((1, m), lambda i: (0, 0))],
        out_specs=pl.BlockSpec((BN, m), lambda i: (i, 0)),
        out_shape=jax.ShapeDtypeStruct((N, m), jnp.float32),
    )(x, w, brow)


def _mid_body(agg_ref, xroot_ref, w_ref, b_ref, h1_ref, o_ref):
    h1 = jnp.maximum(agg_ref[0] + agg_ref[1] + xroot_ref[...], 0.0)
    h1_ref[...] = h1
    o_ref[...] = jnp.dot(h1, w_ref[...],
                         preferred_element_type=jnp.float32) + b_ref[...]


def _tc_mid(aggpair, projall, w, brow):
    k = w.shape[0]
    m = w.shape[1]
    return pl.pallas_call(
        _mid_body,
        grid=(N // BN,),
        in_specs=[pl.BlockSpec((NC, BN, k), lambda i: (0, i, 0)),
                  pl.BlockSpec((BN, k), lambda i: (i, 0)),
                  pl.BlockSpec((k, m), lambda i: (0, 0)),
                  pl.BlockSpec((1, m), lambda i: (0, 0))],
        out_specs=[pl.BlockSpec((BN, k), lambda i: (i, 0)),
                   pl.BlockSpec((BN, m), lambda i: (i, 0))],
        out_shape=[jax.ShapeDtypeStruct((N, k), jnp.float32),
                   jax.ShapeDtypeStruct((N, m), jnp.float32)],
    )(aggpair, projall, w, brow)


def _head_body(h1_ref, agg_ref, hroot_ref, fc1_wT_ref, fc1_b_ref,
               w_ihT_ref, bsum_ref, fc2_wT_ref, fc2_b_ref, o_ref,
               acc1, acc2):
    i = pl.program_id(0)

    @pl.when(i == 0)
    def _():
        acc1[...] = jnp.zeros_like(acc1)
        acc2[...] = jnp.zeros_like(acc2)

    h2 = jnp.maximum(agg_ref[0] + agg_ref[1] + hroot_ref[...], 0.0)
    acc1[...] += jnp.sum(h1_ref[...], axis=0, keepdims=True)
    acc2[...] += jnp.sum(h2, axis=0, keepdims=True)

    @pl.when(i == N // BN - 1)
    def _():
        pooled = jnp.concatenate([acc1[...], acc2[...]], axis=1) / float(N)
        f1 = jnp.maximum(
            jnp.dot(pooled, fc1_wT_ref[...],
                    preferred_element_type=jnp.float32) + fc1_b_ref[...], 0.0)
        gates = jnp.dot(f1, w_ihT_ref[...],
                        preferred_element_type=jnp.float32) + bsum_ref[...]
        hd = 20
        i_g = jax.nn.sigmoid(gates[:, :hd])
        g_g = jnp.tanh(gates[:, 2 * hd:3 * hd])
        o_g = jax.nn.sigmoid(gates[:, 3 * hd:4 * hd])
        hvec = o_g * jnp.tanh(i_g * g_g)
        logits = jnp.dot(hvec, fc2_wT_ref[...],
                         preferred_element_type=jnp.float32) + fc2_b_ref[...]
        m = jnp.max(logits, axis=1, keepdims=True)
        lse = m + jnp.log(jnp.sum(jnp.exp(logits - m), axis=1, keepdims=True))
        o_ref[...] = logits - lse


def _tc_head(h1, aggpair, hroot, fc1_wT, fc1_b, w_ihT, bsum, fc2_wT, fc2_b):
    return pl.pallas_call(
        _head_body,
        grid=(N // BN,),
        in_specs=[pl.BlockSpec((BN, H1), lambda i: (i, 0)),
                  pl.BlockSpec((NC, BN, H2), lambda i: (0, i, 0)),
                  pl.BlockSpec((BN, H2), lambda i: (i, 0)),
                  pl.BlockSpec(fc1_wT.shape, lambda i: (0, 0)),
                  pl.BlockSpec(fc1_b.shape, lambda i: (0, 0)),
                  pl.BlockSpec(w_ihT.shape, lambda i: (0, 0)),
                  pl.BlockSpec(bsum.shape, lambda i: (0, 0)),
                  pl.BlockSpec(fc2_wT.shape, lambda i: (0, 0)),
                  pl.BlockSpec(fc2_b.shape, lambda i: (0, 0))],
        out_specs=pl.BlockSpec((1, 10), lambda i: (0, 0)),
        out_shape=jax.ShapeDtypeStruct((1, 10), jnp.float32),
        scratch_shapes=[pltpu.VMEM((1, H1), jnp.float32),
                        pltpu.VMEM((1, H2), jnp.float32)],
    )(h1, aggpair, hroot, fc1_wT, fc1_b, w_ihT, bsum, fc2_wT, fc2_b)


def kernel(x, edge_index, edge_attr, batch, w1, root1, b1, w2, root2, b2,
           fc1_w, fc1_b, w_ih, w_hh, b_ih, b_hh, fc2_w, fc2_b):
    src = edge_index[0]
    dst = edge_index[1]
    et = edge_attr

    wcat1 = jnp.concatenate(
        [jnp.transpose(w1, (1, 0, 2)).reshape(F_IN, R * H1), root1], axis=1)
    brow1 = jnp.concatenate([jnp.zeros((R * H1,), jnp.float32), b1])[None, :]
    wcat2 = jnp.concatenate(
        [jnp.transpose(w2, (1, 0, 2)).reshape(H1, R * H2), root2], axis=1)
    brow2 = jnp.concatenate([jnp.zeros((R * H2,), jnp.float32), b2])[None, :]

    g, norm = _sc_norm(src, dst, et)

    projall1 = _tc_mm(x, wcat1, brow1)
    table1 = projall1.reshape(N * (R + 1), H1)
    agg1 = _sc_agg(H1, table1, g, dst, norm)

    h1, projall2 = _tc_mid(agg1, projall1[:, R * H1:], wcat2, brow2)
    table2 = projall2.reshape(N * (R + 1), H2)
    agg2 = _sc_agg(H2, table2, g, dst, norm)

    out = _tc_head(h1, agg2, projall2[:, R * H2:],
                   fc1_w.T, fc1_b[None, :], w_ih.T,
                   (b_ih + b_hh)[None, :], fc2_w.T, fc2_b[None, :])
    return out.reshape((10,))
